# baseline (speedup 1.0000x reference)
import functools

import jax
import jax.numpy as jnp
from jax.experimental import pallas as pl
from jax.experimental.pallas import tpu as pltpu

_CHUNK = 2000


def _fused_body(x_hbm, w_ref, b_ref, fc_ref, o_hbm, xs, os, in_sems, out_sems, *, n):
    h = fc_ref.shape[0]
    n_chunks = n // _CHUNK

    def in_copy(i, slot):
        return pltpu.make_async_copy(
            x_hbm.at[pl.ds(i * _CHUNK, _CHUNK), :], xs.at[slot], in_sems.at[slot]
        )

    def out_copy(i, slot):
        return pltpu.make_async_copy(
            os.at[slot], o_hbm.at[pl.ds(i * _CHUNK, _CHUNK), :], out_sems.at[slot]
        )

    in_copy(0, 0).start()
    for i in range(n_chunks):
        slot = i % 2
        if i + 1 < n_chunks:
            in_copy(i + 1, (i + 1) % 2).start()
        in_copy(i, slot).wait()
        pre = (
            jnp.dot(xs[slot], w_ref[:], preferred_element_type=jnp.float32)
            + b_ref[0, : 2 * h]
        )
        th = jnp.tanh(pre)
        g = jnp.maximum((1.0 - th[:, :h]) * th[:, h:], 0.0)
        res = (
            jnp.dot(g, fc_ref[:], preferred_element_type=jnp.float32)
            + b_ref[0, 2 * h]
        )
        if i >= 2:
            out_copy(i - 2, slot).wait()
        os[slot] = res
        out_copy(i, slot).start()
    for i in (n_chunks - 2, n_chunks - 1):
        if i >= 0:
            out_copy(i, i % 2).wait()


def kernel(x, edge_index, edge_weight, Wz, bz, Wr, br, Wh, bh, fc_w, fc_b):
    n, f = x.shape
    h = Wz.shape[-1]
    wz_eff = 0.5 * (Wz[0, 0, :f] + Wz[1, 0, :f])
    wh_eff = Wh[0, 0, :f] + Wh[1, 0, :f]
    w_cat = jnp.concatenate([wz_eff, wh_eff], axis=1)
    b_all = jnp.concatenate([0.5 * bz, bh, fc_b]).reshape(1, 2 * h + 1)
    fc_col = 0.5 * fc_w.reshape(h, 1)

    out = pl.pallas_call(
        functools.partial(_fused_body, n=n),
        in_specs=[
            pl.BlockSpec(memory_space=pl.ANY),
            pl.BlockSpec((f, 2 * h), lambda: (0, 0)),
            pl.BlockSpec((1, 2 * h + 1), lambda: (0, 0)),
            pl.BlockSpec((h, 1), lambda: (0, 0)),
        ],
        out_specs=pl.BlockSpec(memory_space=pl.ANY),
        out_shape=jax.ShapeDtypeStruct((n, 1), x.dtype),
        scratch_shapes=[
            pltpu.VMEM((2, _CHUNK, f), jnp.float32),
            pltpu.VMEM((2, _CHUNK, 1), jnp.float32),
            pltpu.SemaphoreType.DMA((2,)),
            pltpu.SemaphoreType.DMA((2,)),
        ],
    )(x, w_cat, b_all, fc_col)
    return out


# speedup vs baseline: 1.0766x; 1.0766x over previous
import jax
import jax.numpy as jnp
from jax.experimental import pallas as pl
from jax.experimental.pallas import tpu as pltpu

_BLK = 10000


def _fused_body(x_ref, w_ref, b_ref, fc_ref, o_ref):
    h = fc_ref.shape[0]
    pre = (
        jnp.dot(x_ref[:], w_ref[:], preferred_element_type=jnp.float32)
        + b_ref[0, : 2 * h]
    )
    th = jnp.tanh(pre)
    g = jnp.maximum((1.0 - th[:, :h]) * th[:, h:], 0.0)
    o_ref[:] = (
        jnp.dot(g, fc_ref[:], preferred_element_type=jnp.float32) + b_ref[0, 2 * h]
    )


def kernel(x, edge_index, edge_weight, Wz, bz, Wr, br, Wh, bh, fc_w, fc_b):
    n, f = x.shape
    h = Wz.shape[-1]
    wz_eff = 0.5 * (Wz[0, 0, :f] + Wz[1, 0, :f])
    wh_eff = Wh[0, 0, :f] + Wh[1, 0, :f]
    w_cat = jnp.concatenate([wz_eff, wh_eff], axis=1)
    b_all = jnp.concatenate([0.5 * bz, bh, fc_b]).reshape(1, 2 * h + 1)
    fc_col = 0.5 * fc_w.reshape(h, 1)

    grid = (n // _BLK,)
    out = pl.pallas_call(
        _fused_body,
        grid=grid,
        in_specs=[
            pl.BlockSpec((_BLK, f), lambda i: (i, 0)),
            pl.BlockSpec((f, 2 * h), lambda i: (0, 0)),
            pl.BlockSpec((1, 2 * h + 1), lambda i: (0, 0)),
            pl.BlockSpec((h, 1), lambda i: (0, 0)),
        ],
        out_specs=pl.BlockSpec((_BLK, 1), lambda i: (i, 0)),
        out_shape=jax.ShapeDtypeStruct((n, 1), x.dtype),
        compiler_params=pltpu.CompilerParams(
            dimension_semantics=("parallel",),
        ),
    )(x, w_cat, b_all, fc_col)
    return out
